# baseline (speedup 1.0000x reference)
import jax
import jax.numpy as jnp
from jax.experimental import pallas as pl
from jax.experimental.pallas import tpu as pltpu

D_HIDDEN = 1024
D_EXPERT = 256
N_EXPERTS = 8
N_UNITS = 10


def _moe_kernel(x_ref, xb_ref, wr_ref, wg_ref, wu_ref, wd_ref, wsg_ref, wsu_ref,
                wsd_ref, out_ref, probs_ref, idx_ref, aux_ref, comb_scr,
                h_scr, wd_scr):
    e = pl.program_id(0)
    T = x_ref.shape[0]

    @pl.when(e == 0)
    def _router():
        x = x_ref[...]
        logits = jax.lax.dot_general(
            x, wr_ref[...], (((1,), (1,)), ((), ())),
            preferred_element_type=jnp.float32)
        m = jnp.max(logits, axis=1, keepdims=True)
        ex = jnp.exp(logits - m)
        probs = ex / jnp.sum(ex, axis=1, keepdims=True)
        probs_ref[...] = probs
        pm = jnp.mean(probs, axis=0)
        aux_ref[...] = (jnp.float32(N_EXPERTS) * jnp.sum(pm * pm)).reshape(1, 1)
        iota = jax.lax.broadcasted_iota(jnp.int32, (T, N_EXPERTS), 1)
        v1 = jnp.max(probs, axis=1, keepdims=True)
        i1 = jnp.min(jnp.where(probs == v1, iota, N_EXPERTS), axis=1, keepdims=True)
        masked = jnp.where(iota == i1, -jnp.inf, probs)
        v2 = jnp.max(masked, axis=1, keepdims=True)
        i2 = jnp.min(jnp.where(masked == v2, iota, N_EXPERTS), axis=1, keepdims=True)
        idx_ref[...] = jnp.concatenate([i1, i2], axis=1)
        s = v1 + v2
        w1 = v1 / s
        w2 = v2 / s
        lanes = comb_scr.shape[1]
        iota_u = jax.lax.broadcasted_iota(jnp.int32, (T, lanes), 1)
        comb = (jnp.where(iota_u == i1, w1, 0.0)
                + jnp.where(iota_u == i2, w2, 0.0))
        comb_scr[...] = comb.astype(jnp.bfloat16)

    @pl.when(e < N_EXPERTS)
    def _expert_step():
        lanes = comb_scr.shape[1]
        iota_u = jax.lax.broadcasted_iota(jnp.int32, (T, lanes), 1)
        w_col = jnp.sum(jnp.where(iota_u == e,
                                  comb_scr[...].astype(jnp.float32), 0.0),
                        axis=1, keepdims=True)

        xb = xb_ref[...]
        wg = wg_ref[0].astype(jnp.bfloat16)
        wu = wu_ref[0].astype(jnp.bfloat16)

        g = jax.lax.dot_general(xb, wg, (((1,), (1,)), ((), ())),
                                preferred_element_type=jnp.float32)
        u = jax.lax.dot_general(xb, wu, (((1,), (1,)), ((), ())),
                                preferred_element_type=jnp.float32)
        h = (g * jax.nn.sigmoid(g) * u * w_col).astype(jnp.bfloat16)
        wdb = wd_ref[0].astype(jnp.bfloat16)
        for us in range(N_EXPERTS):
            @pl.when(e == us)
            def _store():
                h_scr[:, us * D_EXPERT:(us + 1) * D_EXPERT] = h
                wd_scr[:, us * D_EXPERT:(us + 1) * D_EXPERT] = wdb

        @pl.when(e < 2)
        def _with_shared():
            wsg = wsg_ref[0].astype(jnp.bfloat16)
            wsu = wsu_ref[0].astype(jnp.bfloat16)
            gs = jax.lax.dot_general(xb, wsg, (((1,), (1,)), ((), ())),
                                     preferred_element_type=jnp.float32)
            us2 = jax.lax.dot_general(xb, wsu, (((1,), (1,)), ((), ())),
                                      preferred_element_type=jnp.float32)
            hs = (gs * jax.nn.sigmoid(gs) * us2).astype(jnp.bfloat16)
            wsdb = wsd_ref[0].astype(jnp.bfloat16)
            for us in range(2):
                @pl.when(e == us)
                def _store_shared():
                    lo = (N_EXPERTS + us) * D_EXPERT
                    h_scr[:, lo:lo + D_EXPERT] = hs
                    wd_scr[:, lo:lo + D_EXPERT] = wsdb

    @pl.when(e >= N_EXPERTS)
    def _reduce():
        r = e - N_EXPERTS
        hrows = h_scr[pl.ds(r * (T // N_EXPERTS), T // N_EXPERTS), :]
        out_ref[...] = jax.lax.dot_general(
            hrows, wd_scr[...], (((1,), (1,)), ((), ())),
            preferred_element_type=jnp.float32)


def kernel(x, W_g, Wg_e, Wu_e, Wd_e, Ws_g, Ws_u, Ws_d):
    B, S, D = x.shape
    T = B * S
    x_flat = x.reshape(T, D)
    ws_g2 = Ws_g.reshape(2, D_EXPERT, D)
    ws_u2 = Ws_u.reshape(2, D_EXPERT, D)
    ws_d2 = Ws_d.reshape(D, 2, D_EXPERT).transpose(1, 0, 2)

    grid = (2 * N_EXPERTS,)
    out, probs, idx, aux = pl.pallas_call(
        _moe_kernel,
        grid=grid,
        in_specs=[
            pl.BlockSpec((T, D), lambda e: (0, 0)),
            pl.BlockSpec((T, D), lambda e: (0, 0)),
            pl.BlockSpec((N_EXPERTS, D), lambda e: (0, 0)),
            pl.BlockSpec((1, D_EXPERT, D),
                         lambda e: (jnp.minimum(e, N_EXPERTS - 1), 0, 0)),
            pl.BlockSpec((1, D_EXPERT, D),
                         lambda e: (jnp.minimum(e, N_EXPERTS - 1), 0, 0)),
            pl.BlockSpec((1, D, D_EXPERT),
                         lambda e: (jnp.minimum(e, N_EXPERTS - 1), 0, 0)),
            pl.BlockSpec((1, D_EXPERT, D),
                         lambda e: (jnp.minimum(e, 1), 0, 0)),
            pl.BlockSpec((1, D_EXPERT, D),
                         lambda e: (jnp.minimum(e, 1), 0, 0)),
            pl.BlockSpec((1, D, D_EXPERT),
                         lambda e: (jnp.minimum(e, 1), 0, 0)),
        ],
        out_specs=[
            pl.BlockSpec((T // N_EXPERTS, D),
                         lambda e: (jnp.maximum(e - N_EXPERTS, 0), 0)),
            pl.BlockSpec((T, N_EXPERTS), lambda e: (0, 0)),
            pl.BlockSpec((T, 2), lambda e: (0, 0)),
            pl.BlockSpec((1, 1), lambda e: (0, 0)),
        ],
        out_shape=[
            jax.ShapeDtypeStruct((T, D), jnp.float32),
            jax.ShapeDtypeStruct((T, N_EXPERTS), jnp.float32),
            jax.ShapeDtypeStruct((T, 2), jnp.int32),
            jax.ShapeDtypeStruct((1, 1), jnp.float32),
        ],
        scratch_shapes=[
            pltpu.VMEM((T, 128), jnp.bfloat16),
            pltpu.VMEM((T, N_UNITS * D_EXPERT), jnp.bfloat16),
            pltpu.VMEM((D_HIDDEN, N_UNITS * D_EXPERT), jnp.bfloat16),
        ],
        compiler_params=pltpu.CompilerParams(
            dimension_semantics=("arbitrary",),
        ),
    )(x_flat, x_flat.astype(jnp.bfloat16), W_g, Wg_e, Wu_e, Wd_e,
      ws_g2, ws_u2, ws_d2)

    return (out.reshape(B, S, D), probs.reshape(B, S, N_EXPERTS),
            idx.reshape(B, S, 2), aux.reshape(()))


# speedup vs baseline: 1.0204x; 1.0204x over previous
import jax
import jax.numpy as jnp
from jax.experimental import pallas as pl
from jax.experimental.pallas import tpu as pltpu

D_HIDDEN = 1024
D_EXPERT = 256
N_EXPERTS = 8
N_UNITS = 10
N_REDUCE = 4


def _moe_kernel(x_ref, xb_ref, wr_ref, wg_ref, wu_ref, wd_ref, wsg_ref, wsu_ref,
                wsd_ref, out_ref, probs_ref, idx_ref, aux_ref, comb_scr,
                h_scr, wd_scr):
    e = pl.program_id(0)
    T = x_ref.shape[0]

    @pl.when(e == 0)
    def _router():
        x = x_ref[...]
        logits = jax.lax.dot_general(
            x, wr_ref[...], (((1,), (1,)), ((), ())),
            preferred_element_type=jnp.float32)
        m = jnp.max(logits, axis=1, keepdims=True)
        ex = jnp.exp(logits - m)
        probs = ex / jnp.sum(ex, axis=1, keepdims=True)
        probs_ref[...] = probs
        pm = jnp.mean(probs, axis=0)
        aux_ref[...] = (jnp.float32(N_EXPERTS) * jnp.sum(pm * pm)).reshape(1, 1)
        iota = jax.lax.broadcasted_iota(jnp.int32, (T, N_EXPERTS), 1)
        v1 = jnp.max(probs, axis=1, keepdims=True)
        i1 = jnp.min(jnp.where(probs == v1, iota, N_EXPERTS), axis=1, keepdims=True)
        masked = jnp.where(iota == i1, -jnp.inf, probs)
        v2 = jnp.max(masked, axis=1, keepdims=True)
        i2 = jnp.min(jnp.where(masked == v2, iota, N_EXPERTS), axis=1, keepdims=True)
        idx_ref[...] = jnp.concatenate([i1, i2], axis=1)
        s = v1 + v2
        w1 = v1 / s
        w2 = v2 / s
        lanes = comb_scr.shape[1]
        iota_u = jax.lax.broadcasted_iota(jnp.int32, (T, lanes), 1)
        comb = (jnp.where(iota_u == i1, w1, 0.0)
                + jnp.where(iota_u == i2, w2, 0.0))
        comb_scr[...] = comb.astype(jnp.bfloat16)

    @pl.when(e < N_EXPERTS)
    def _expert_step():
        lanes = comb_scr.shape[1]
        iota_u = jax.lax.broadcasted_iota(jnp.int32, (T, lanes), 1)
        w_col = jnp.sum(jnp.where(iota_u == e,
                                  comb_scr[...].astype(jnp.float32), 0.0),
                        axis=1, keepdims=True)

        xb = xb_ref[...]
        wg = wg_ref[0].astype(jnp.bfloat16)
        wu = wu_ref[0].astype(jnp.bfloat16)

        g = jax.lax.dot_general(xb, wg, (((1,), (1,)), ((), ())),
                                preferred_element_type=jnp.float32)
        u = jax.lax.dot_general(xb, wu, (((1,), (1,)), ((), ())),
                                preferred_element_type=jnp.float32)
        h = (g * jax.nn.sigmoid(g) * u * w_col).astype(jnp.bfloat16)
        wdb = wd_ref[0].astype(jnp.bfloat16)
        for us in range(N_EXPERTS):
            @pl.when(e == us)
            def _store():
                h_scr[:, us * D_EXPERT:(us + 1) * D_EXPERT] = h
                wd_scr[:, us * D_EXPERT:(us + 1) * D_EXPERT] = wdb

        @pl.when(e < 2)
        def _with_shared():
            wsg = wsg_ref[0].astype(jnp.bfloat16)
            wsu = wsu_ref[0].astype(jnp.bfloat16)
            gs = jax.lax.dot_general(xb, wsg, (((1,), (1,)), ((), ())),
                                     preferred_element_type=jnp.float32)
            us2 = jax.lax.dot_general(xb, wsu, (((1,), (1,)), ((), ())),
                                      preferred_element_type=jnp.float32)
            hs = (gs * jax.nn.sigmoid(gs) * us2).astype(jnp.bfloat16)
            wsdb = wsd_ref[0].astype(jnp.bfloat16)
            for us in range(2):
                @pl.when(e == us)
                def _store_shared():
                    lo = (N_EXPERTS + us) * D_EXPERT
                    h_scr[:, lo:lo + D_EXPERT] = hs
                    wd_scr[:, lo:lo + D_EXPERT] = wsdb

    @pl.when(e >= N_EXPERTS)
    def _reduce():
        r = e - N_EXPERTS
        nb = D_HIDDEN // N_REDUCE
        wdrows = wd_scr[pl.ds(r * nb, nb), :]
        out_ref[...] = jax.lax.dot_general(
            h_scr[...], wdrows, (((1,), (1,)), ((), ())),
            preferred_element_type=jnp.float32)


def kernel(x, W_g, Wg_e, Wu_e, Wd_e, Ws_g, Ws_u, Ws_d):
    B, S, D = x.shape
    T = B * S
    x_flat = x.reshape(T, D)
    ws_g2 = Ws_g.reshape(2, D_EXPERT, D)
    ws_u2 = Ws_u.reshape(2, D_EXPERT, D)
    ws_d2 = Ws_d.reshape(D, 2, D_EXPERT).transpose(1, 0, 2)

    grid = (N_EXPERTS + N_REDUCE,)
    out, probs, idx, aux = pl.pallas_call(
        _moe_kernel,
        grid=grid,
        in_specs=[
            pl.BlockSpec((T, D), lambda e: (0, 0)),
            pl.BlockSpec((T, D), lambda e: (0, 0)),
            pl.BlockSpec((N_EXPERTS, D), lambda e: (0, 0)),
            pl.BlockSpec((1, D_EXPERT, D),
                         lambda e: (jnp.minimum(e, N_EXPERTS - 1), 0, 0)),
            pl.BlockSpec((1, D_EXPERT, D),
                         lambda e: (jnp.minimum(e, N_EXPERTS - 1), 0, 0)),
            pl.BlockSpec((1, D, D_EXPERT),
                         lambda e: (jnp.minimum(e, N_EXPERTS - 1), 0, 0)),
            pl.BlockSpec((1, D_EXPERT, D),
                         lambda e: (jnp.minimum(e, 1), 0, 0)),
            pl.BlockSpec((1, D_EXPERT, D),
                         lambda e: (jnp.minimum(e, 1), 0, 0)),
            pl.BlockSpec((1, D, D_EXPERT),
                         lambda e: (jnp.minimum(e, 1), 0, 0)),
        ],
        out_specs=[
            pl.BlockSpec((T, D // N_REDUCE),
                         lambda e: (0, jnp.maximum(e - N_EXPERTS, 0))),
            pl.BlockSpec((T, N_EXPERTS), lambda e: (0, 0)),
            pl.BlockSpec((T, 2), lambda e: (0, 0)),
            pl.BlockSpec((1, 1), lambda e: (0, 0)),
        ],
        out_shape=[
            jax.ShapeDtypeStruct((T, D), jnp.float32),
            jax.ShapeDtypeStruct((T, N_EXPERTS), jnp.float32),
            jax.ShapeDtypeStruct((T, 2), jnp.int32),
            jax.ShapeDtypeStruct((1, 1), jnp.float32),
        ],
        scratch_shapes=[
            pltpu.VMEM((T, 128), jnp.bfloat16),
            pltpu.VMEM((T, N_UNITS * D_EXPERT), jnp.bfloat16),
            pltpu.VMEM((D_HIDDEN, N_UNITS * D_EXPERT), jnp.bfloat16),
        ],
        compiler_params=pltpu.CompilerParams(
            dimension_semantics=("arbitrary",),
        ),
    )(x_flat, x_flat.astype(jnp.bfloat16), W_g, Wg_e, Wu_e, Wd_e,
      ws_g2, ws_u2, ws_d2)

    return (out.reshape(B, S, D), probs.reshape(B, S, N_EXPERTS),
            idx.reshape(B, S, 2), aux.reshape(()))


# speedup vs baseline: 1.0414x; 1.0205x over previous
import jax
import jax.numpy as jnp
from jax.experimental import pallas as pl
from jax.experimental.pallas import tpu as pltpu

D_HIDDEN = 1024
D_EXPERT = 256
N_EXPERTS = 8
N_UNITS = 10


def _moe_kernel(x_ref, wr_ref, wg_ref, wu_ref, wd_ref, wsg_ref, wsu_ref, wsd_ref,
                out_ref, probs_ref, idx_ref, aux_ref, xb_scr, comb_scr, y_scr):
    e = pl.program_id(0)
    T = x_ref.shape[0]

    @pl.when(e == 0)
    def _router():
        x = x_ref[...]
        xb_scr[...] = x.astype(jnp.bfloat16)
        logits = jax.lax.dot_general(
            x, wr_ref[...], (((1,), (1,)), ((), ())),
            preferred_element_type=jnp.float32)
        m = jnp.max(logits, axis=1, keepdims=True)
        ex = jnp.exp(logits - m)
        probs = ex / jnp.sum(ex, axis=1, keepdims=True)
        probs_ref[...] = probs
        pm = jnp.mean(probs, axis=0)
        aux_ref[...] = (jnp.float32(N_EXPERTS) * jnp.sum(pm * pm)).reshape(1, 1)
        iota = jax.lax.broadcasted_iota(jnp.int32, (T, N_EXPERTS), 1)
        v1 = jnp.max(probs, axis=1, keepdims=True)
        i1 = jnp.min(jnp.where(probs == v1, iota, N_EXPERTS), axis=1, keepdims=True)
        masked = jnp.where(iota == i1, -jnp.inf, probs)
        v2 = jnp.max(masked, axis=1, keepdims=True)
        i2 = jnp.min(jnp.where(masked == v2, iota, N_EXPERTS), axis=1, keepdims=True)
        idx_ref[...] = jnp.concatenate([i1, i2], axis=1)
        s = v1 + v2
        w1 = v1 / s
        w2 = v2 / s
        lanes = comb_scr.shape[1]
        iota_u = jax.lax.broadcasted_iota(jnp.int32, (T, lanes), 1)
        comb = (jnp.where(iota_u == i1, w1, 0.0)
                + jnp.where(iota_u == i2, w2, 0.0))
        comb_scr[...] = comb

    @pl.when(e == 1)
    def _fold_first():
        out_ref[...] = y_scr[...].astype(jnp.float32)

    @pl.when(e > 1)
    def _fold():
        out_ref[...] = out_ref[...] + y_scr[...].astype(jnp.float32)

    lanes = comb_scr.shape[1]
    iota_u = jax.lax.broadcasted_iota(jnp.int32, (T, lanes), 1)
    w_col = jnp.sum(jnp.where(iota_u == e, comb_scr[...], 0.0),
                    axis=1, keepdims=True).astype(jnp.bfloat16)

    xb = xb_scr[...]
    wg = wg_ref[0].astype(jnp.bfloat16)
    wu = wu_ref[0].astype(jnp.bfloat16)
    wd = wd_ref[0].astype(jnp.bfloat16)

    g = jax.lax.dot_general(xb, wg, (((1,), (1,)), ((), ())),
                            preferred_element_type=jnp.float32)
    u = jax.lax.dot_general(xb, wu, (((1,), (1,)), ((), ())),
                            preferred_element_type=jnp.float32)
    h = (g * jax.nn.sigmoid(g)).astype(jnp.bfloat16) * u.astype(jnp.bfloat16) * w_col
    y = jax.lax.dot_general(h, wd, (((1,), (1,)), ((), ())),
                            preferred_element_type=jnp.float32)

    @pl.when(e < 2)
    def _with_shared():
        wsg = wsg_ref[0].astype(jnp.bfloat16)
        wsu = wsu_ref[0].astype(jnp.bfloat16)
        wsd = wsd_ref[0].astype(jnp.bfloat16)
        gs = jax.lax.dot_general(xb, wsg, (((1,), (1,)), ((), ())),
                                 preferred_element_type=jnp.float32)
        us = jax.lax.dot_general(xb, wsu, (((1,), (1,)), ((), ())),
                                 preferred_element_type=jnp.float32)
        hs = (gs * jax.nn.sigmoid(gs)).astype(jnp.bfloat16) * us.astype(jnp.bfloat16)
        ys = jax.lax.dot_general(hs, wsd, (((1,), (1,)), ((), ())),
                                 preferred_element_type=jnp.float32)
        y_scr[...] = (y + ys).astype(jnp.bfloat16)

    @pl.when(e >= 2)
    def _plain():
        y_scr[...] = y.astype(jnp.bfloat16)

    @pl.when(e == N_EXPERTS - 1)
    def _last():
        out_ref[...] = out_ref[...] + y_scr[...].astype(jnp.float32)


def kernel(x, W_g, Wg_e, Wu_e, Wd_e, Ws_g, Ws_u, Ws_d):
    B, S, D = x.shape
    T = B * S
    x_flat = x.reshape(T, D)
    ws_g2 = Ws_g.reshape(2, D_EXPERT, D)
    ws_u2 = Ws_u.reshape(2, D_EXPERT, D)
    ws_d2 = Ws_d.reshape(D, 2, D_EXPERT).transpose(1, 0, 2)

    grid = (N_EXPERTS,)
    out, probs, idx, aux = pl.pallas_call(
        _moe_kernel,
        grid=grid,
        in_specs=[
            pl.BlockSpec((T, D), lambda e: (0, 0)),
            pl.BlockSpec((N_EXPERTS, D), lambda e: (0, 0)),
            pl.BlockSpec((1, D_EXPERT, D), lambda e: (e, 0, 0)),
            pl.BlockSpec((1, D_EXPERT, D), lambda e: (e, 0, 0)),
            pl.BlockSpec((1, D, D_EXPERT), lambda e: (e, 0, 0)),
            pl.BlockSpec((1, D_EXPERT, D),
                         lambda e: (jnp.minimum(e, 1), 0, 0)),
            pl.BlockSpec((1, D_EXPERT, D),
                         lambda e: (jnp.minimum(e, 1), 0, 0)),
            pl.BlockSpec((1, D, D_EXPERT),
                         lambda e: (jnp.minimum(e, 1), 0, 0)),
        ],
        out_specs=[
            pl.BlockSpec((T, D), lambda e: (0, 0)),
            pl.BlockSpec((T, N_EXPERTS), lambda e: (0, 0)),
            pl.BlockSpec((T, 2), lambda e: (0, 0)),
            pl.BlockSpec((1, 1), lambda e: (0, 0)),
        ],
        out_shape=[
            jax.ShapeDtypeStruct((T, D), jnp.float32),
            jax.ShapeDtypeStruct((T, N_EXPERTS), jnp.float32),
            jax.ShapeDtypeStruct((T, 2), jnp.int32),
            jax.ShapeDtypeStruct((1, 1), jnp.float32),
        ],
        scratch_shapes=[
            pltpu.VMEM((T, D_HIDDEN), jnp.bfloat16),
            pltpu.VMEM((T, 128), jnp.float32),
            pltpu.VMEM((T, D_HIDDEN), jnp.bfloat16),
        ],
        compiler_params=pltpu.CompilerParams(
            dimension_semantics=("arbitrary",),
        ),
    )(x_flat, W_g, Wg_e, Wu_e, Wd_e, ws_g2, ws_u2, ws_d2)

    return (out.reshape(B, S, D), probs.reshape(B, S, N_EXPERTS),
            idx.reshape(B, S, 2), aux.reshape(()))
